# baseline (speedup 1.0000x reference)
import functools

import jax
import jax.numpy as jnp
from jax import lax
from jax.experimental import pallas as pl
from jax.experimental.pallas import tpu as pltpu
from jax.experimental.pallas import tpu_sc as plsc

_NC = 2
_NS = 16
_NW = _NC * _NS
_L = 16

_B = 128
_GPW = 80
_NSL = 6


def _half_tc(x):
    def body(x_ref, o_ref):
        o_ref[...] = x_ref[...] * 0.5

    N, D = x.shape
    return pl.pallas_call(
        body,
        out_shape=jax.ShapeDtypeStruct((N, D), jnp.float32),
        grid=(10,),
        in_specs=[pl.BlockSpec((N // 10, D), lambda i: (i, 0))],
        out_specs=pl.BlockSpec((N // 10, D), lambda i: (i, 0)),
    )(x)


@functools.lru_cache(maxsize=None)
def _graph_pool_sc(N, M, D):
    NCHUNK = M // _B
    RW = (N // _NW) & ~7
    NTAIL = N - _NW * RW
    mesh = plsc.VectorSubcoreMesh(core_axis_name="c", subcore_axis_name="s")

    @functools.partial(
        pl.kernel,
        out_type=jax.ShapeDtypeStruct((N + M, D), jnp.float32),
        mesh=mesh,
        scratch_types=[
            pltpu.VMEM((_GPW, _B), jnp.int32),
            pltpu.VMEM((_GPW, _B), jnp.int32),
        ]
        + [pltpu.VMEM((_B, D), jnp.float32) for _ in range(_NSL)]
        + [pltpu.SemaphoreType.DMA for _ in range(2 * _NSL)],
    )
    def k(x_hbm, xh_hbm, i0_hbm, i1_hbm, out_hbm, ia, ib, *bufs):
        rs = bufs[:_NSL]
        gsem = bufs[_NSL:2 * _NSL]
        osem = bufs[2 * _NSL:3 * _NSL]
        cid = lax.axis_index("c")
        sid = lax.axis_index("s")
        wid = cid * _NS + sid

        pltpu.sync_copy(x_hbm.at[pl.ds(wid * RW, RW)],
                        out_hbm.at[pl.ds(wid * RW, RW)])

        @pl.when(wid == 0)
        def _copy_top_tail():
            pltpu.sync_copy(x_hbm.at[pl.ds(_NW * RW, NTAIL)],
                            out_hbm.at[pl.ds(_NW * RW, NTAIL)])

        row0 = wid * _GPW
        nv = jnp.minimum(_GPW, jnp.maximum(0, NCHUNK - row0))
        pltpu.sync_copy(i0_hbm.at[pl.ds(row0, _GPW)], ia)
        pltpu.sync_copy(i1_hbm.at[pl.ds(row0, _GPW)], ib)

        def plain(g, s):
            pltpu.async_copy(xh_hbm.at[ia.at[g]], rs[s], gsem[s])

        def wait_g(g, s):
            pltpu.make_async_copy(xh_hbm.at[ia.at[g]], rs[s], gsem[s]).wait()

        def addg(g, s):
            pltpu.async_copy(xh_hbm.at[ib.at[g]], rs[s], gsem[s], add=True)

        def drain_out(s):
            pltpu.make_async_copy(rs[s], out_hbm.at[pl.ds(N, _B)],
                                  osem[s]).wait()

        for g0 in range(_NSL - 2):
            plain(g0, g0)
        wait_g(0, 0)
        addg(0, 0)

        def step(t, carry):
            for sub in range(_NSL):
                g = t * _NSL + sub

                @pl.when(g < nv)
                def _do():
                    s0 = sub
                    s1 = (sub + 1) % _NSL
                    s4 = (sub + _NSL - 2) % _NSL

                    @pl.when(g + _NSL - 2 < nv)
                    def _issue_plain():
                        @pl.when(g >= 2)
                        def _drain():
                            drain_out(s4)
                        plain(g + _NSL - 2, s4)

                    @pl.when(g + 1 < nv)
                    def _issue_add():
                        wait_g(g + 1, s1)
                        addg(g + 1, s1)

                    wait_g(g, s0)
                    r = row0 + g
                    pltpu.async_copy(
                        rs[s0], out_hbm.at[pl.ds(N + r * _B, _B)], osem[s0])

            return carry

        lax.fori_loop(0, (nv + _NSL - 1) // _NSL, step, 0)
        for s in range(_NSL):
            drain_out(s)

    return k


@jax.jit
def kernel(x, pool_idx):
    N, D = x.shape
    M = pool_idx.shape[0]
    npad = _NW * _GPW * _B - M
    pad = jnp.arange(npad, dtype=jnp.int32) % N
    i0 = jnp.concatenate([pool_idx[:, 0], pad]).reshape(_NW * _GPW, _B)
    i1 = jnp.concatenate([pool_idx[:, 1], pad]).reshape(_NW * _GPW, _B)
    xh = _half_tc(x)
    return _graph_pool_sc(N, M, D)(x, xh, i0, i1)


# speedup vs baseline: 1.0047x; 1.0047x over previous
import functools

import jax
import jax.numpy as jnp
from jax import lax
from jax.experimental import pallas as pl
from jax.experimental.pallas import tpu as pltpu
from jax.experimental.pallas import tpu_sc as plsc

_NC = 2
_NS = 16
_NW = _NC * _NS
_L = 16

_B = 128
_GPW = 80
_NS_SLOTS = 5


def _half_tc(x):
    def body(x_ref, o_ref, o2_ref):
        h = x_ref[...] * 0.5
        o_ref[...] = h
        o2_ref[...] = h

    N, D = x.shape
    return pl.pallas_call(
        body,
        out_shape=[jax.ShapeDtypeStruct((N, D), jnp.float32)] * 2,
        grid=(10,),
        in_specs=[pl.BlockSpec((N // 10, D), lambda i: (i, 0))],
        out_specs=[pl.BlockSpec((N // 10, D), lambda i: (i, 0))] * 2,
    )(x)


@functools.lru_cache(maxsize=None)
def _graph_pool_sc(N, M, D):
    NCHUNK = M // _B
    RW = (N // _NW) & ~7
    NTAIL = N - _NW * RW
    mesh = plsc.VectorSubcoreMesh(core_axis_name="c", subcore_axis_name="s")

    @functools.partial(
        pl.kernel,
        out_type=jax.ShapeDtypeStruct((N + M, D), jnp.float32),
        mesh=mesh,
        scratch_types=[
            pltpu.VMEM((_GPW, _B), jnp.int32),
            pltpu.VMEM((_GPW, _B), jnp.int32),
        ]
        + [pltpu.VMEM((_B, D), jnp.float32) for _ in range(_NS_SLOTS)]
        + [pltpu.SemaphoreType.DMA for _ in range(2 * _NS_SLOTS)],
    )
    def k(x_hbm, xh_hbm, xh2_hbm, i0_hbm, i1_hbm, out_hbm, ia, ib, *bufs):
        rs = bufs[:_NS_SLOTS]
        gsem = bufs[_NS_SLOTS:2 * _NS_SLOTS]
        osem = bufs[2 * _NS_SLOTS:3 * _NS_SLOTS]
        cid = lax.axis_index("c")
        sid = lax.axis_index("s")
        wid = cid * _NS + sid

        pltpu.sync_copy(x_hbm.at[pl.ds(wid * RW, RW)],
                        out_hbm.at[pl.ds(wid * RW, RW)])

        @pl.when(wid == 0)
        def _copy_top_tail():
            pltpu.sync_copy(x_hbm.at[pl.ds(_NW * RW, NTAIL)],
                            out_hbm.at[pl.ds(_NW * RW, NTAIL)])

        row0 = wid * _GPW
        nv = jnp.minimum(_GPW, jnp.maximum(0, NCHUNK - row0))
        pltpu.sync_copy(i0_hbm.at[pl.ds(row0, _GPW)], ia)
        pltpu.sync_copy(i1_hbm.at[pl.ds(row0, _GPW)], ib)

        def plain(g, s):
            pltpu.async_copy(xh_hbm.at[ia.at[g]], rs[s], gsem[s])

        def wait_g(g, s):
            pltpu.make_async_copy(xh_hbm.at[ia.at[g]], rs[s], gsem[s]).wait()

        def addg(g, s):
            pltpu.async_copy(xh2_hbm.at[ib.at[g]], rs[s], gsem[s], add=True)

        def drain_out(s):
            pltpu.make_async_copy(rs[s], out_hbm.at[pl.ds(N, _B)],
                                  osem[s]).wait()

        plain(0, 0)
        plain(1, 1)
        plain(2, 2)
        wait_g(0, 0)
        addg(0, 0)

        def step(t, carry):
            for sub in range(_NS_SLOTS):
                g = t * _NS_SLOTS + sub
                s0 = sub
                s1 = (sub + 1) % _NS_SLOTS
                s3 = (sub + 3) % _NS_SLOTS

                @pl.when(g + 3 < nv)
                def _issue_plain():
                    @pl.when(g >= 2)
                    def _drain():
                        drain_out(s3)
                    plain(g + 3, s3)

                @pl.when(g + 1 < nv)
                def _issue_add():
                    wait_g(g + 1, s1)
                    addg(g + 1, s1)

                wait_g(g, s0)
                r = row0 + g
                pltpu.async_copy(rs[s0], out_hbm.at[pl.ds(N + r * _B, _B)],
                                 osem[s0])
            return carry

        lax.fori_loop(0, nv // _NS_SLOTS, step, 0)
        for s in range(_NS_SLOTS):
            drain_out(s)

    return k


@jax.jit
def kernel(x, pool_idx):
    N, D = x.shape
    M = pool_idx.shape[0]
    npad = _NW * _GPW * _B - M
    pad = jnp.arange(npad, dtype=jnp.int32) % N
    i0 = jnp.concatenate([pool_idx[:, 0], pad]).reshape(_NW * _GPW, _B)
    i1 = jnp.concatenate([pool_idx[:, 1], pad]).reshape(_NW * _GPW, _B)
    xh, xh2 = _half_tc(x)
    return _graph_pool_sc(N, M, D)(x, xh, xh2, i0, i1)


# speedup vs baseline: 1.0050x; 1.0003x over previous
import functools

import jax
import jax.numpy as jnp
from jax import lax
from jax.experimental import pallas as pl
from jax.experimental.pallas import tpu as pltpu
from jax.experimental.pallas import tpu_sc as plsc

_NC = 2
_NS = 16
_NW = _NC * _NS
_L = 16

_B = 128
_GPW = 80
_NS_SLOTS = 5


def _half_tc(x):
    def body(x_ref, o_ref):
        o_ref[...] = x_ref[...] * 0.5

    N, D = x.shape
    return pl.pallas_call(
        body,
        out_shape=jax.ShapeDtypeStruct((N, D), jnp.float32),
        grid=(10,),
        in_specs=[pl.BlockSpec((N // 10, D), lambda i: (i, 0))],
        out_specs=pl.BlockSpec((N // 10, D), lambda i: (i, 0)),
    )(x)


@functools.lru_cache(maxsize=None)
def _graph_pool_sc(N, M, D):
    NCHUNK = M // _B
    RW = (N // _NW) & ~7
    NTAIL = N - _NW * RW
    mesh = plsc.VectorSubcoreMesh(core_axis_name="c", subcore_axis_name="s")

    @functools.partial(
        pl.kernel,
        out_type=jax.ShapeDtypeStruct((N + M, D), jnp.float32),
        mesh=mesh,
        scratch_types=[
            pltpu.VMEM((_GPW, _B), jnp.int32),
            pltpu.VMEM((_GPW, _B), jnp.int32),
        ]
        + [pltpu.VMEM((_B, D), jnp.float32) for _ in range(_NS_SLOTS)]
        + [pltpu.SemaphoreType.DMA for _ in range(2 * _NS_SLOTS)],
    )
    def k(x_hbm, xh_hbm, i0_hbm, i1_hbm, out_hbm, ia, ib, *bufs):
        rs = bufs[:_NS_SLOTS]
        gsem = bufs[_NS_SLOTS:2 * _NS_SLOTS]
        osem = bufs[2 * _NS_SLOTS:3 * _NS_SLOTS]
        cid = lax.axis_index("c")
        sid = lax.axis_index("s")
        wid = cid * _NS + sid

        pltpu.sync_copy(x_hbm.at[pl.ds(wid * RW, RW)],
                        out_hbm.at[pl.ds(wid * RW, RW)])

        @pl.when(wid == 0)
        def _copy_top_tail():
            pltpu.sync_copy(x_hbm.at[pl.ds(_NW * RW, NTAIL)],
                            out_hbm.at[pl.ds(_NW * RW, NTAIL)])

        row0 = wid * _GPW
        nv = jnp.minimum(_GPW, jnp.maximum(0, NCHUNK - row0))
        pltpu.sync_copy(i0_hbm.at[pl.ds(row0, _GPW)], ia)
        pltpu.sync_copy(i1_hbm.at[pl.ds(row0, _GPW)], ib)

        def plain(g, s):
            pltpu.async_copy(xh_hbm.at[ia.at[g]], rs[s], gsem[s])

        def wait_g(g, s):
            pltpu.make_async_copy(xh_hbm.at[ia.at[g]], rs[s], gsem[s]).wait()

        def addg(g, s):
            pltpu.async_copy(xh_hbm.at[ib.at[g]], rs[s], gsem[s], add=True)

        def drain_out(s):
            pltpu.make_async_copy(rs[s], out_hbm.at[pl.ds(N, _B)],
                                  osem[s]).wait()

        plain(0, 0)
        plain(1, 1)
        plain(2, 2)
        wait_g(0, 0)
        addg(0, 0)

        def step(t, carry):
            for sub in range(_NS_SLOTS):
                g = t * _NS_SLOTS + sub
                s0 = sub
                s1 = (sub + 1) % _NS_SLOTS
                s3 = (sub + 3) % _NS_SLOTS

                @pl.when(g + 3 < nv)
                def _issue_plain():
                    @pl.when(g >= 2)
                    def _drain():
                        drain_out(s3)
                    plain(g + 3, s3)

                @pl.when(g + 1 < nv)
                def _issue_add():
                    wait_g(g + 1, s1)
                    addg(g + 1, s1)

                wait_g(g, s0)
                r = row0 + g
                pltpu.async_copy(rs[s0], out_hbm.at[pl.ds(N + r * _B, _B)],
                                 osem[s0])
            return carry

        lax.fori_loop(0, nv // _NS_SLOTS, step, 0)
        for s in range(_NS_SLOTS):
            drain_out(s)

    return k


@jax.jit
def kernel(x, pool_idx):
    N, D = x.shape
    M = pool_idx.shape[0]
    npad = _NW * _GPW * _B - M
    pad = jnp.arange(npad, dtype=jnp.int32) % N
    i0 = jnp.concatenate([pool_idx[:, 0], pad]).reshape(_NW * _GPW, _B)
    i1 = jnp.concatenate([pool_idx[:, 1], pad]).reshape(_NW * _GPW, _B)
    xh = _half_tc(x)
    return _graph_pool_sc(N, M, D)(x, xh, i0, i1)
